# baseline (speedup 1.0000x reference)
import functools

import jax
import jax.numpy as jnp
from jax.experimental import pallas as pl

_BIG = 1e30


def _chamfer_body(g_ref, p_ref, pbias_ref, m_ref, c_ref,
                  out_p_ref, out_g_ref, *, num_chunks, tj):
    b = pl.program_id(0)

    P = p_ref[0]
    pbias = pbias_ref[0]

    step_g = jnp.zeros((), jnp.float32)
    pmin = None
    for c in range(num_chunks):
        G = g_ref[0, c * tj:(c + 1) * tj, :]
        gn = jnp.sum(G * G, axis=1, keepdims=True)
        E = jax.lax.dot_general(G * (-2.0), P, (((1,), (1,)), ((), ())),
                                preferred_element_type=jnp.float32)

        gmin = gn + jnp.min(E + pbias, axis=1, keepdims=True)
        step_g += jnp.sum(jnp.maximum(gmin, 0.0))

        cmin = jnp.min(E + gn, axis=0, keepdims=True)
        pmin = cmin if pmin is None else jnp.minimum(pmin, cmin)

    m = m_ref[0]
    conf = c_ref[0]
    safe_conf = jnp.where(m > 0, conf, 1.0)
    loss_p = (jnp.maximum(pmin + pbias, 0.0) * conf * m
              - jnp.log(safe_conf) * m)
    step_p = jnp.sum(loss_p)

    @pl.when(b == 0)
    def _():
        out_p_ref[...] = jnp.zeros_like(out_p_ref)
        out_g_ref[...] = jnp.zeros_like(out_g_ref)

    out_p_ref[...] += jnp.full((1, 1), step_p, jnp.float32)
    out_g_ref[...] += jnp.full((1, 1), step_g, jnp.float32)


def kernel(x_gt, x_pred, mask, confidence):
    B, V1, _ = x_gt.shape
    V2 = x_pred.shape[1]
    TJ = 256
    num_chunks = V1 // TJ

    m = jnp.squeeze(mask, -1).astype(jnp.float32)
    pn = jnp.sum(x_pred * x_pred, axis=-1)
    pbias = pn + (1.0 - m) * _BIG

    out_p, out_g = pl.pallas_call(
        functools.partial(_chamfer_body, num_chunks=num_chunks, tj=TJ),
        grid=(B,),
        in_specs=[
            pl.BlockSpec((1, V1, 3), lambda b: (b, 0, 0)),
            pl.BlockSpec((1, V2, 3), lambda b: (b, 0, 0)),
            pl.BlockSpec((1, 1, V2), lambda b: (b, 0, 0)),
            pl.BlockSpec((1, 1, V2), lambda b: (b, 0, 0)),
            pl.BlockSpec((1, 1, V2), lambda b: (b, 0, 0)),
        ],
        out_specs=[
            pl.BlockSpec((1, 1), lambda b: (0, 0)),
            pl.BlockSpec((1, 1), lambda b: (0, 0)),
        ],
        out_shape=[
            jax.ShapeDtypeStruct((1, 1), jnp.float32),
            jax.ShapeDtypeStruct((1, 1), jnp.float32),
        ],
    )(x_gt, x_pred, pbias[:, None, :], m[:, None, :], confidence[:, None, :])

    return out_p[0, 0] / (B * V2) + out_g[0, 0] / (B * V1)


# speedup vs baseline: 1.0121x; 1.0121x over previous
import functools

import jax
import jax.numpy as jnp
from jax.experimental import pallas as pl

_BIG = 1e30


def _chamfer_body(g_ref, p_ref, pbias_ref, m_ref, c_ref,
                  out_p_ref, out_g_ref, *, num_chunks, tj):
    b = pl.program_id(0)

    P = p_ref[0]
    pbias = pbias_ref[0]

    step_g = jnp.zeros((), jnp.float32)
    pmin = None
    for c in range(num_chunks):
        G = g_ref[0, c * tj:(c + 1) * tj, :]
        gn = jnp.sum(G * G, axis=1, keepdims=True)
        E = jax.lax.dot_general(G * (-2.0), P, (((1,), (1,)), ((), ())),
                                preferred_element_type=jnp.float32)

        gmin = gn + jnp.min(E + pbias, axis=1, keepdims=True)
        step_g += jnp.sum(jnp.maximum(gmin, 0.0))

        cmin = jnp.min(E + gn, axis=0, keepdims=True)
        pmin = cmin if pmin is None else jnp.minimum(pmin, cmin)

    m = m_ref[0]
    conf = c_ref[0]
    safe_conf = jnp.where(m > 0, conf, 1.0)
    loss_p = (jnp.maximum(pmin + pbias, 0.0) * conf * m
              - jnp.log(safe_conf) * m)
    step_p = jnp.sum(loss_p)

    @pl.when(b == 0)
    def _():
        out_p_ref[...] = jnp.zeros_like(out_p_ref)
        out_g_ref[...] = jnp.zeros_like(out_g_ref)

    out_p_ref[...] += jnp.full((1, 1), step_p, jnp.float32)
    out_g_ref[...] += jnp.full((1, 1), step_g, jnp.float32)


def kernel(x_gt, x_pred, mask, confidence):
    B, V1, _ = x_gt.shape
    V2 = x_pred.shape[1]
    TJ = 512
    num_chunks = V1 // TJ

    m = jnp.squeeze(mask, -1).astype(jnp.float32)
    pn = jnp.sum(x_pred * x_pred, axis=-1)
    pbias = pn + (1.0 - m) * _BIG

    out_p, out_g = pl.pallas_call(
        functools.partial(_chamfer_body, num_chunks=num_chunks, tj=TJ),
        grid=(B,),
        in_specs=[
            pl.BlockSpec((1, V1, 3), lambda b: (b, 0, 0)),
            pl.BlockSpec((1, V2, 3), lambda b: (b, 0, 0)),
            pl.BlockSpec((1, 1, V2), lambda b: (b, 0, 0)),
            pl.BlockSpec((1, 1, V2), lambda b: (b, 0, 0)),
            pl.BlockSpec((1, 1, V2), lambda b: (b, 0, 0)),
        ],
        out_specs=[
            pl.BlockSpec((1, 1), lambda b: (0, 0)),
            pl.BlockSpec((1, 1), lambda b: (0, 0)),
        ],
        out_shape=[
            jax.ShapeDtypeStruct((1, 1), jnp.float32),
            jax.ShapeDtypeStruct((1, 1), jnp.float32),
        ],
    )(x_gt, x_pred, pbias[:, None, :], m[:, None, :], confidence[:, None, :])

    return out_p[0, 0] / (B * V2) + out_g[0, 0] / (B * V1)
